# baseline (speedup 1.0000x reference)
import functools

import jax
import jax.numpy as jnp
from jax import lax
from jax.experimental import pallas as pl
from jax.experimental.pallas import tpu as pltpu
from jax.experimental.pallas import tpu_sc as plsc

N = 10000
E = 320000
D = 128

NC = 2
NS = 16
NW = NC * NS
CHUNK = 88
CHUNKS = 114
EPT = CHUNKS * CHUNK
EPAD = NW * EPT
ROWS = 10112
RPT = ROWS // NS
PADROW = 10016
HR = 80
NBUF = 4
HC = 6
GROUPS = 19

_mesh = plsc.VectorSubcoreMesh(core_axis_name="c", subcore_axis_name="s")
_sc_params = pltpu.CompilerParams(needs_layout_passes=False)


@functools.partial(
    pl.kernel,
    out_type=jax.ShapeDtypeStruct((NW, HR, D), jnp.float32),
    mesh=_mesh,
    scratch_types=[
        pltpu.VMEM((EPT,), jnp.int32),
        pltpu.VMEM((HR, D), jnp.float32),
    ],
    compiler_params=_sc_params,
)
def _deg_kernel(dst_hbm, zeros_hbm, parts_hbm, dstv, hist):
    c = lax.axis_index("c")
    s = lax.axis_index("s")
    w = c * NS + s
    pltpu.sync_copy(dst_hbm.at[w], dstv)
    pltpu.sync_copy(zeros_hbm.at[pl.ds(0, HR)], hist)
    ones = jnp.ones((16,), jnp.float32)

    def body(i, carry):
        idx = dstv[pl.ds(i * 16, 16)]
        plsc.addupdate_scatter(hist, [idx >> 7, idx & 127], ones)
        return carry

    lax.fori_loop(0, EPT // 16, body, 0)
    pltpu.sync_copy(hist, parts_hbm.at[w])


@functools.partial(
    pl.kernel,
    out_type=jax.ShapeDtypeStruct((NC, ROWS, D), jnp.float32),
    mesh=_mesh,
    scratch_types=[
        [pltpu.VMEM((HC, CHUNK), jnp.int32) for _ in range(2)],
        [pltpu.VMEM((HC, CHUNK), jnp.int32) for _ in range(2)],
        [pltpu.VMEM((CHUNK, D), jnp.float32) for _ in range(NBUF)],
        [pltpu.SemaphoreType.DMA for _ in range(NBUF)],
        [pltpu.SemaphoreType.DMA for _ in range(NBUF)],
        [pltpu.SemaphoreType.DMA for _ in range(2)],
        [pltpu.SemaphoreType.DMA for _ in range(2)],
        pltpu.VMEM_SHARED((ROWS, D), jnp.float32),
    ],
    compiler_params=_sc_params,
)
def _prop_kernel(u_hbm, src_hbm, dst_hbm, zeros_hbm, parts_hbm,
                 srcv2, dstv2, bufs, gsems, ssems, isrc, idst, acc):
    c = lax.axis_index("c")
    s = lax.axis_index("s")
    w = c * NS + s

    def load_idx(h, p):
        pltpu.async_copy(src_hbm.at[w * GROUPS + h], srcv2[p], isrc[p])
        pltpu.async_copy(dst_hbm.at[w * GROUPS + h], dstv2[p], idst[p])

    def wait_idx(h, p):
        pltpu.make_async_copy(src_hbm.at[w * GROUPS + h], srcv2[p],
                              isrc[p]).wait()
        pltpu.make_async_copy(dst_hbm.at[w * GROUPS + h], dstv2[p],
                              idst[p]).wait()

    load_idx(0, 0)
    pltpu.sync_copy(zeros_hbm, acc.at[pl.ds(s * RPT, RPT)])
    plsc.subcore_barrier()

    SD = NBUF - 2

    def do_group(h, p, last):
        srcv, dstv = srcv2[p], dstv2[p]
        wait_idx(h, p)
        if not last:
            load_idx(h + 1, 1 - p)
        for b in range(2):
            pltpu.async_copy(u_hbm.at[srcv.at[b]], bufs[b], gsems[b])
        for j in range(HC):
            b = j % NBUF
            pltpu.make_async_copy(u_hbm.at[srcv.at[j]], bufs[b],
                                  gsems[b]).wait()
            pltpu.async_copy(bufs[b], acc.at[dstv.at[j]], ssems[b],
                             add=True)
            if j + 2 < HC:
                bb = (j + 2) % NBUF
                if j >= SD:
                    pltpu.make_async_copy(bufs[bb], acc.at[dstv.at[j - SD]],
                                          ssems[bb]).wait()
                pltpu.async_copy(u_hbm.at[srcv.at[j + 2]], bufs[bb],
                                 gsems[bb])
        for k in range(max(HC - NBUF, 0), HC):
            pltpu.make_async_copy(bufs[k % NBUF], acc.at[dstv.at[k]],
                                  ssems[k % NBUF]).wait()

    def pair(h0, carry):
        do_group(2 * h0, 0, False)
        do_group(2 * h0 + 1, 1, False)
        return carry

    lax.fori_loop(0, GROUPS // 2, pair, 0)
    do_group(GROUPS - 1, 0, True)
    plsc.subcore_barrier()
    pltpu.sync_copy(acc.at[pl.ds(s * RPT, RPT)],
                    parts_hbm.at[c, pl.ds(s * RPT, RPT)])


def _prepscale_body(p_ref, x_ref, u0_ref, dis_ref, dinv_ref):
    deg = jnp.sum(p_ref[:, 0], axis=0) + 1.0
    dis = lax.rsqrt(deg)
    dinv = 1.0 / deg
    r = lax.broadcasted_iota(jnp.int32, (D, D), 0)
    cc = lax.broadcasted_iota(jnp.int32, (D, D), 1)
    eye = jnp.where(r == cc, 1.0, 0.0).astype(jnp.float32)
    dn = (((1,), (1,)), ((), ()))
    dis_col = lax.dot_general(eye, dis, dn, preferred_element_type=jnp.float32)
    dinv_col = lax.dot_general(eye, dinv, dn, preferred_element_type=jnp.float32)
    u0_ref[...] = x_ref[...] * dis_col
    dis_ref[...] = dis_col
    dinv_ref[...] = dinv_col


_prepscale = pl.pallas_call(
    _prepscale_body,
    grid=(ROWS // D,),
    in_specs=[pl.BlockSpec((NW, 1, 1, D), lambda i: (0, i, 0, 0)),
              pl.BlockSpec((D, D), lambda i: (i, 0))],
    out_specs=[pl.BlockSpec((D, D), lambda i: (i, 0)),
               pl.BlockSpec((D, 1), lambda i: (i, 0)),
               pl.BlockSpec((D, 1), lambda i: (i, 0))],
    out_shape=[jax.ShapeDtypeStruct((ROWS, D), jnp.float32),
               jax.ShapeDtypeStruct((ROWS, 1), jnp.float32),
               jax.ShapeDtypeStruct((ROWS, 1), jnp.float32)],
)

_BR = 1264


def _comb_body(p_ref, u_ref, s_ref, o_ref):
    o_ref[...] = (p_ref[0] + p_ref[1] + u_ref[...]) * s_ref[...]


_comb = pl.pallas_call(
    _comb_body,
    grid=(ROWS // _BR,),
    in_specs=[pl.BlockSpec((2, _BR, D), lambda i: (0, i, 0)),
              pl.BlockSpec((_BR, D), lambda i: (i, 0)),
              pl.BlockSpec((_BR, 1), lambda i: (i, 0))],
    out_specs=pl.BlockSpec((_BR, D), lambda i: (i, 0)),
    out_shape=jax.ShapeDtypeStruct((ROWS, D), jnp.float32),
)


def _final_body(p_ref, u_ref, s_ref, w_ref, b_ref, o_ref):
    h = (p_ref[0] + p_ref[1] + u_ref[...]) * s_ref[...]
    o_ref[...] = lax.dot_general(
        h, w_ref[...], (((1,), (1,)), ((), ())),
        preferred_element_type=jnp.float32) + b_ref[...]


_final = pl.pallas_call(
    _final_body,
    grid=(ROWS // _BR,),
    in_specs=[pl.BlockSpec((2, _BR, D), lambda i: (0, i, 0)),
              pl.BlockSpec((_BR, D), lambda i: (i, 0)),
              pl.BlockSpec((_BR, 1), lambda i: (i, 0)),
              pl.BlockSpec((D, D), lambda i: (0, 0)),
              pl.BlockSpec((1, D), lambda i: (0, 0))],
    out_specs=pl.BlockSpec((_BR, D), lambda i: (i, 0)),
    out_shape=jax.ShapeDtypeStruct((ROWS, D), jnp.float32),
)


def kernel(x, edge_index, W, b):
    ei = edge_index.astype(jnp.int32)
    pad = jnp.full((EPAD - E,), PADROW, jnp.int32)
    src = jnp.concatenate([ei[0], pad])
    dst = jnp.concatenate([ei[1], pad])
    src3 = src.reshape(NW * GROUPS, HC, CHUNK)
    dst3 = dst.reshape(NW * GROUPS, HC, CHUNK)
    dstf = dst.reshape(NW, EPT)
    zeros2d = jnp.zeros((RPT, D), jnp.float32)

    deg_parts = _deg_kernel(dstf, zeros2d)
    x_pad = jnp.concatenate([x, jnp.zeros((ROWS - N, D), jnp.float32)])
    u0, dis_col, dinv_col = _prepscale(
        deg_parts.reshape(NW, HR, 1, D), x_pad)
    parts1 = _prop_kernel(u0, src3, dst3, zeros2d)
    u1 = _comb(parts1, u0, dinv_col)
    parts2 = _prop_kernel(u1, src3, dst3, zeros2d)
    out = _final(parts2, u1, dis_col, W, b.reshape(1, D))
    return out[:N]


# speedup vs baseline: 1.0552x; 1.0552x over previous
import functools

import jax
import jax.numpy as jnp
from jax import lax
from jax.experimental import pallas as pl
from jax.experimental.pallas import tpu as pltpu
from jax.experimental.pallas import tpu_sc as plsc

N = 10000
E = 320000
D = 128

NC = 2
NS = 16
NW = NC * NS
CHUNK = 88
CHUNKS = 114
EPT = CHUNKS * CHUNK
EPAD = NW * EPT
ROWS = 10112
RPT = ROWS // NS
PADROW = 10016
HR = 80
NBUF = 4
HC = 6
GROUPS = 19

_mesh = plsc.VectorSubcoreMesh(core_axis_name="c", subcore_axis_name="s")
_sc_params = pltpu.CompilerParams(needs_layout_passes=False)


@functools.partial(
    pl.kernel,
    out_type=jax.ShapeDtypeStruct((NW, HR, D), jnp.float32),
    mesh=_mesh,
    scratch_types=[
        pltpu.VMEM((EPT,), jnp.int32),
        pltpu.VMEM((HR, D), jnp.float32),
    ],
    compiler_params=_sc_params,
)
def _deg_kernel(dst_hbm, zeros_hbm, parts_hbm, dstv, hist):
    c = lax.axis_index("c")
    s = lax.axis_index("s")
    w = c * NS + s
    pltpu.sync_copy(dst_hbm.at[w], dstv)
    pltpu.sync_copy(zeros_hbm.at[pl.ds(0, HR)], hist)
    ones = jnp.ones((16,), jnp.float32)

    def body(i, carry):
        idx = dstv[pl.ds(i * 16, 16)]
        plsc.addupdate_scatter(hist, [idx >> 7, idx & 127], ones)
        return carry

    lax.fori_loop(0, EPT // 16, body, 0)
    pltpu.sync_copy(hist, parts_hbm.at[w])


@functools.partial(
    pl.kernel,
    out_type=jax.ShapeDtypeStruct((NC, ROWS, D), jnp.float32),
    mesh=_mesh,
    scratch_types=[
        [pltpu.VMEM((HC, CHUNK), jnp.int32) for _ in range(2)],
        [pltpu.VMEM((HC, CHUNK), jnp.int32) for _ in range(2)],
        [pltpu.VMEM((CHUNK, D), jnp.float32) for _ in range(NBUF)],
        [pltpu.SemaphoreType.DMA for _ in range(NBUF)],
        [pltpu.SemaphoreType.DMA for _ in range(NBUF)],
        [pltpu.SemaphoreType.DMA for _ in range(2)],
        [pltpu.SemaphoreType.DMA for _ in range(2)],
        pltpu.VMEM_SHARED((ROWS, D), jnp.float32),
    ],
    compiler_params=_sc_params,
)
def _prop_kernel(u_hbm, src_hbm, dst_hbm, zeros_hbm, parts_hbm,
                 srcv2, dstv2, bufs, gsems, ssems, isrc, idst, acc):
    c = lax.axis_index("c")
    s = lax.axis_index("s")
    w = c * NS + s

    def load_idx(h, p):
        pltpu.async_copy(src_hbm.at[w * GROUPS + h], srcv2[p], isrc[p])
        pltpu.async_copy(dst_hbm.at[w * GROUPS + h], dstv2[p], idst[p])

    def wait_idx(h, p):
        pltpu.make_async_copy(src_hbm.at[w * GROUPS + h], srcv2[p],
                              isrc[p]).wait()
        pltpu.make_async_copy(dst_hbm.at[w * GROUPS + h], dstv2[p],
                              idst[p]).wait()

    load_idx(0, 0)
    pltpu.sync_copy(zeros_hbm, acc.at[pl.ds(s * RPT, RPT)])
    plsc.subcore_barrier()

    SD = NBUF - 2

    def do_group(h, p, last):
        srcv, dstv = srcv2[p], dstv2[p]
        wait_idx(h, p)
        if not last:
            load_idx(h + 1, 1 - p)
        for b in range(2):
            pltpu.async_copy(u_hbm.at[srcv.at[b]], bufs[b], gsems[b])
        for j in range(HC):
            b = j % NBUF
            pltpu.make_async_copy(u_hbm.at[srcv.at[j]], bufs[b],
                                  gsems[b]).wait()
            pltpu.async_copy(bufs[b], acc.at[dstv.at[j]], ssems[b],
                             add=True)
            if j + 2 < HC:
                bb = (j + 2) % NBUF
                if j >= SD:
                    pltpu.make_async_copy(bufs[bb], acc.at[dstv.at[j - SD]],
                                          ssems[bb]).wait()
                pltpu.async_copy(u_hbm.at[srcv.at[j + 2]], bufs[bb],
                                 gsems[bb])
        for k in range(max(HC - NBUF, 0), HC):
            pltpu.make_async_copy(bufs[k % NBUF], acc.at[dstv.at[k]],
                                  ssems[k % NBUF]).wait()

    def pair(h0, carry):
        do_group(2 * h0, 0, False)
        do_group(2 * h0 + 1, 1, False)
        return carry

    lax.fori_loop(0, GROUPS // 2, pair, 0)
    do_group(GROUPS - 1, 0, True)
    plsc.subcore_barrier()
    pltpu.sync_copy(acc.at[pl.ds(s * RPT, RPT)],
                    parts_hbm.at[c, pl.ds(s * RPT, RPT)])


def _prep_body(p_ref, dis_ref, dinv_ref):
    deg = jnp.sum(p_ref[...], axis=0) + 1.0
    dis_ref[...] = lax.rsqrt(deg)
    dinv_ref[...] = 1.0 / deg


_prep = pl.pallas_call(
    _prep_body,
    out_shape=[jax.ShapeDtypeStruct((HR, D), jnp.float32),
               jax.ShapeDtypeStruct((HR, D), jnp.float32)],
)

_BR = 1264


def _scale_body(x_ref, s_ref, o_ref):
    o_ref[...] = x_ref[...] * s_ref[...]


_scale = pl.pallas_call(
    _scale_body,
    grid=(ROWS // _BR,),
    in_specs=[pl.BlockSpec((_BR, D), lambda i: (i, 0)),
              pl.BlockSpec((_BR, 1), lambda i: (i, 0))],
    out_specs=pl.BlockSpec((_BR, D), lambda i: (i, 0)),
    out_shape=jax.ShapeDtypeStruct((ROWS, D), jnp.float32),
)


def _comb_body(p_ref, u_ref, s_ref, o_ref):
    o_ref[...] = (p_ref[0] + p_ref[1] + u_ref[...]) * s_ref[...]


_comb = pl.pallas_call(
    _comb_body,
    grid=(ROWS // _BR,),
    in_specs=[pl.BlockSpec((2, _BR, D), lambda i: (0, i, 0)),
              pl.BlockSpec((_BR, D), lambda i: (i, 0)),
              pl.BlockSpec((_BR, 1), lambda i: (i, 0))],
    out_specs=pl.BlockSpec((_BR, D), lambda i: (i, 0)),
    out_shape=jax.ShapeDtypeStruct((ROWS, D), jnp.float32),
)


def _final_body(p_ref, u_ref, s_ref, w_ref, b_ref, o_ref):
    h = (p_ref[0] + p_ref[1] + u_ref[...]) * s_ref[...]
    o_ref[...] = lax.dot_general(
        h, w_ref[...], (((1,), (1,)), ((), ())),
        preferred_element_type=jnp.float32) + b_ref[...]


_final = pl.pallas_call(
    _final_body,
    grid=(ROWS // _BR,),
    in_specs=[pl.BlockSpec((2, _BR, D), lambda i: (0, i, 0)),
              pl.BlockSpec((_BR, D), lambda i: (i, 0)),
              pl.BlockSpec((_BR, 1), lambda i: (i, 0)),
              pl.BlockSpec((D, D), lambda i: (0, 0)),
              pl.BlockSpec((1, D), lambda i: (0, 0))],
    out_specs=pl.BlockSpec((_BR, D), lambda i: (i, 0)),
    out_shape=jax.ShapeDtypeStruct((ROWS, D), jnp.float32),
)


def kernel(x, edge_index, W, b):
    ei = edge_index.astype(jnp.int32)
    pad = jnp.full((EPAD - E,), PADROW, jnp.int32)
    src = jnp.concatenate([ei[0], pad])
    dst = jnp.concatenate([ei[1], pad])
    src3 = src.reshape(NW * GROUPS, HC, CHUNK)
    dst3 = dst.reshape(NW * GROUPS, HC, CHUNK)
    dstf = dst.reshape(NW, EPT)
    zeros2d = jnp.zeros((RPT, D), jnp.float32)

    deg_parts = _deg_kernel(dstf, zeros2d)
    dis2d, dinv2d = _prep(deg_parts)
    dis_col = dis2d.reshape(HR * D, 1)[:ROWS]
    dinv_col = dinv2d.reshape(HR * D, 1)[:ROWS]

    x_pad = jnp.concatenate([x, jnp.zeros((ROWS - N, D), jnp.float32)])
    u0 = _scale(x_pad, dis_col)
    parts1 = _prop_kernel(u0, src3, dst3, zeros2d)
    u1 = _comb(parts1, u0, dinv_col)
    parts2 = _prop_kernel(u1, src3, dst3, zeros2d)
    out = _final(parts2, u1, dis_col, W, b.reshape(1, D))
    return out[:N]
